# baseline (speedup 1.0000x reference)
import functools

import jax
import jax.numpy as jnp
from jax import lax
from jax.experimental import pallas as pl
from jax.experimental.pallas import tpu as pltpu
from jax.experimental.pallas import tpu_sc as plsc

VOCAB = 1000000
EMBED_DIM = 64
BATCH = 16384

NUM_CORES = 2
NUM_SUBCORES = 16
LANES = 16

NW = NUM_CORES * NUM_SUBCORES
B_PER_W = BATCH // NW


def _sc_body(tgt_idx_hbm, ctx_idx_hbm, in_emb_hbm, out_emb_hbm, score_hbm,
             tgt_idx_v, ctx_idx_v, tgt_rows_v, ctx_rows_v, score_v,
             sem_t, sem_c):
    wid = lax.axis_index("s") * NUM_CORES + lax.axis_index("c")
    base = wid * B_PER_W

    pltpu.sync_copy(tgt_idx_hbm.at[pl.ds(base, B_PER_W)], tgt_idx_v)
    pltpu.sync_copy(ctx_idx_hbm.at[pl.ds(base, B_PER_W)], ctx_idx_v)

    cp_t = pltpu.async_copy(in_emb_hbm.at[tgt_idx_v], tgt_rows_v, sem_t)
    cp_c = pltpu.async_copy(out_emb_hbm.at[ctx_idx_v], ctx_rows_v, sem_c)
    cp_t.wait()
    cp_c.wait()

    lane_iota = lax.iota(jnp.int32, LANES)

    def group(g, carry):
        sums = jnp.zeros((LANES,), jnp.float32)
        for r in range(LANES):
            i = g * LANES + r
            acc = jnp.zeros((LANES,), jnp.float32)
            for d in range(0, EMBED_DIM, LANES):
                tv = tgt_rows_v[i, pl.ds(d, LANES)]
                cv = ctx_rows_v[i, pl.ds(d, LANES)]
                acc = acc + tv * cv
            sums = jnp.where(lane_iota == r, jnp.sum(acc), sums)
        score_v[pl.ds(g * LANES, LANES)] = sums
        return carry

    lax.fori_loop(0, B_PER_W // LANES, group, 0)

    pltpu.sync_copy(score_v, score_hbm.at[pl.ds(base, B_PER_W)])


@jax.jit
def _w2v_scores(tgt_idx, ctx_idx, in_emb, out_emb):
    mesh = plsc.VectorSubcoreMesh(
        core_axis_name="c", subcore_axis_name="s",
        num_cores=NUM_CORES, num_subcores=NUM_SUBCORES)
    return pl.kernel(
        _sc_body,
        out_type=jax.ShapeDtypeStruct((BATCH,), jnp.float32),
        mesh=mesh,
        scratch_types=[
            pltpu.VMEM((B_PER_W,), jnp.int32),
            pltpu.VMEM((B_PER_W,), jnp.int32),
            pltpu.VMEM((B_PER_W, EMBED_DIM), jnp.float32),
            pltpu.VMEM((B_PER_W, EMBED_DIM), jnp.float32),
            pltpu.VMEM((B_PER_W,), jnp.float32),
            pltpu.SemaphoreType.DMA,
            pltpu.SemaphoreType.DMA,
        ],
        compiler_params=pltpu.CompilerParams(
            needs_layout_passes=False, use_tc_tiling_on_sc=False),
    )(tgt_idx, ctx_idx, in_emb, out_emb)


def kernel(target_word_idx, context_word_idx, in_emb, out_emb):
    tgt = target_word_idx.astype(jnp.int32)
    ctx = context_word_idx.astype(jnp.int32)
    return _w2v_scores(tgt, ctx, in_emb, out_emb)
